# baseline (speedup 1.0000x reference)
import functools

import jax
import jax.numpy as jnp
from jax import lax
from jax.experimental import pallas as pl
from jax.experimental.pallas import tpu as pltpu
from jax.experimental.pallas import tpu_sc as plsc

EMBED_DIM = 128
SPATIAL = 32 * 32
BB = 8
NBUF = 8


def _make_sc_gather(B, V):
    info = plsc.get_sparse_core_info()
    nw = info.num_cores * info.num_subcores
    b_per_w = B // nw
    mesh = plsc.VectorSubcoreMesh(core_axis_name="c", subcore_axis_name="s")

    @functools.partial(
        pl.kernel, mesh=mesh,
        out_type=jax.ShapeDtypeStruct((B, EMBED_DIM), jnp.float32),
        scratch_types=[
            pltpu.VMEM((b_per_w,), jnp.int32),
            pltpu.VMEM((b_per_w, EMBED_DIM), jnp.float32),
            pltpu.SemaphoreType.DMA,
        ],
    )
    def sc_gather(t_hbm, emb_hbm, out_hbm, idx_v, rows_v, sem):
        wid = lax.axis_index("s") * info.num_cores + lax.axis_index("c")
        base = wid * b_per_w
        pltpu.sync_copy(t_hbm.at[pl.ds(base, b_per_w)], idx_v)
        pltpu.async_copy(emb_hbm.at[idx_v], rows_v, sem).wait()
        pltpu.sync_copy(rows_v, out_hbm.at[pl.ds(base, b_per_w)])

    return sc_gather


def _broadcast_body(g_ref, o_hbm, scratch, sems):
    i = pl.program_id(0)
    nsteps = pl.num_programs(0)
    for k in range(NBUF):
        chunk = i * NBUF + k
        buf = scratch.at[pl.ds(k * BB, BB)]
        cp = pltpu.make_async_copy(
            buf, o_hbm.at[pl.ds(chunk * BB, BB)], sems.at[k])

        @pl.when(i > 0)
        def _():
            cp.wait()

        gt = jnp.swapaxes(g_ref[pl.ds(chunk * BB, BB), :], 0, 1)
        for j in range(BB):
            scratch[k * BB + j] = jnp.broadcast_to(
                gt[:, j:j + 1], (EMBED_DIM, SPATIAL))
        cp.start()

    @pl.when(i == nsteps - 1)
    def _():
        for k in range(NBUF):
            pltpu.make_async_copy(
                scratch.at[pl.ds(k * BB, BB)],
                o_hbm.at[pl.ds(k * BB, BB)], sems.at[k]).wait()


def kernel(x, t, embedding):
    B = t.shape[0]
    V = embedding.shape[0]

    g = _make_sc_gather(B, V)(t, embedding)

    out = pl.pallas_call(
        _broadcast_body,
        grid=(B // (BB * NBUF),),
        in_specs=[pl.BlockSpec((B, EMBED_DIM), lambda i: (0, 0))],
        out_specs=pl.BlockSpec(memory_space=pl.ANY),
        out_shape=jax.ShapeDtypeStruct((B, EMBED_DIM, SPATIAL), jnp.float32),
        scratch_shapes=[
            pltpu.VMEM((NBUF * BB, EMBED_DIM, SPATIAL), jnp.float32),
            pltpu.SemaphoreType.DMA((NBUF,)),
        ],
        compiler_params=pltpu.CompilerParams(
            dimension_semantics=("arbitrary",)),
    )(g)
    return out.reshape(B, EMBED_DIM, x.shape[-2], x.shape[-1])


# speedup vs baseline: 3.5522x; 3.5522x over previous
import functools

import jax
import jax.numpy as jnp
from jax import lax
from jax.experimental import pallas as pl
from jax.experimental.pallas import tpu as pltpu
from jax.experimental.pallas import tpu_sc as plsc

EMBED_DIM = 128
SPATIAL = 32 * 32
BB = 8


def _make_sc_gather(B, V):
    info = plsc.get_sparse_core_info()
    nw = info.num_cores * info.num_subcores
    b_per_w = B // nw
    mesh = plsc.VectorSubcoreMesh(core_axis_name="c", subcore_axis_name="s")

    @functools.partial(
        pl.kernel, mesh=mesh,
        out_type=jax.ShapeDtypeStruct((B, EMBED_DIM), jnp.float32),
        scratch_types=[
            pltpu.VMEM((b_per_w,), jnp.int32),
            pltpu.VMEM((b_per_w, EMBED_DIM), jnp.float32),
            pltpu.SemaphoreType.DMA,
        ],
    )
    def sc_gather(t_hbm, emb_hbm, out_hbm, idx_v, rows_v, sem):
        wid = lax.axis_index("s") * info.num_cores + lax.axis_index("c")
        base = wid * b_per_w
        pltpu.sync_copy(t_hbm.at[pl.ds(base, b_per_w)], idx_v)
        pltpu.async_copy(emb_hbm.at[idx_v], rows_v, sem).wait()
        pltpu.sync_copy(rows_v, out_hbm.at[pl.ds(base, b_per_w)])

    return sc_gather


def _broadcast_body(g_ref, o_ref):
    o_ref[...] = jnp.broadcast_to(
        g_ref[...][:, None, :], (BB, SPATIAL, EMBED_DIM))


def kernel(x, t, embedding):
    B = t.shape[0]
    V = embedding.shape[0]
    H, W = x.shape[-2], x.shape[-1]

    g = _make_sc_gather(B, V)(t, embedding)

    out = pl.pallas_call(
        _broadcast_body,
        grid=(B // BB,),
        in_specs=[pl.BlockSpec((BB, EMBED_DIM), lambda i: (i, 0))],
        out_specs=pl.BlockSpec((BB, SPATIAL, EMBED_DIM), lambda i: (i, 0, 0)),
        out_shape=jax.ShapeDtypeStruct((B, SPATIAL, EMBED_DIM), jnp.float32),
    )(g)
    return out.reshape(B, H, W, EMBED_DIM).transpose(0, 3, 1, 2)
